# baseline (speedup 1.0000x reference)
import functools

import jax
import jax.numpy as jnp
import numpy as np
from jax import lax
from jax.experimental import pallas as pl
from jax.experimental.pallas import tpu as pltpu
from jax.experimental.pallas import tpu_sc as plsc

_NC = 2
_NS = 16
_NW = _NC * _NS
_STEPS = 3
_LP = 128
_WS = 48
_C = 125
_Q = 5


def _pad_cols(x, width):
    n = x.shape[0]
    return jnp.concatenate([x, jnp.zeros((n, width - x.shape[1]), x.dtype)], axis=1)


def _dense_relu_pad(x, w, b):
    n, _ = x.shape

    def body(x_ref, w_ref, b_ref, o_ref):
        acc = jnp.dot(x_ref[...], w_ref[...], preferred_element_type=jnp.float32)
        o_ref[...] = _pad_cols(jnp.maximum(acc + b_ref[...], 0.0),
                               _LP).astype(jnp.bfloat16)

    return pl.pallas_call(
        body,
        out_shape=jax.ShapeDtypeStruct((n, _LP), jnp.bfloat16),
    )(x, w, b)


def _edge_h(e_feat, eW1, eb1, tile):
    e, de = e_feat.shape
    d = eW1.shape[1]

    def body(ef_ref, w_ref, b_ref, o_ref):
        acc = jnp.dot(ef_ref[...], w_ref[...], preferred_element_type=jnp.float32)
        h = jnp.maximum(acc + b_ref[...], 0.0)
        ones = jnp.ones((tile, 1), jnp.float32)
        o_ref[...] = jnp.concatenate(
            [h, ones, jnp.zeros((tile, _LP - d - 1), jnp.float32)],
            axis=1).astype(jnp.bfloat16)

    return pl.pallas_call(
        body,
        grid=(e // tile,),
        in_specs=[
            pl.BlockSpec((tile, de), lambda i: (i, 0)),
            pl.BlockSpec((de, d), lambda i: (0, 0)),
            pl.BlockSpec((1, d), lambda i: (0, 0)),
        ],
        out_specs=pl.BlockSpec((tile, _LP), lambda i: (i, 0)),
        out_shape=jax.ShapeDtypeStruct((e, _LP), jnp.bfloat16),
    )(e_feat, eW1, eb1)


def _edge_messages(h_p, xs_p, t2aug, d, tile):
    e = h_p.shape[0]
    ka = t2aug.shape[0]

    ra_np = np.zeros((_LP, ka), np.float32)
    rb_np = np.zeros((_LP, ka), np.float32)
    eye_d = np.eye(d, dtype=np.float32)
    for k in range(d):
        ra_np[k, k * d:(k + 1) * d] = 1.0
        rb_np[:d, k * d:(k + 1) * d] = eye_d
    ra_np[d, d * d: d * d + d] = 1.0
    rb_np[:d, d * d: d * d + d] = eye_d
    ra = jnp.asarray(ra_np, dtype=jnp.bfloat16)
    rb = jnp.asarray(rb_np, dtype=jnp.bfloat16)

    def body(h_ref, xs_ref, ra_ref, rb_ref, t2_ref, o_ref):
        he = jnp.dot(h_ref[...], ra_ref[...],
                     preferred_element_type=jnp.float32).astype(jnp.bfloat16)
        xt = jnp.dot(xs_ref[...], rb_ref[...],
                     preferred_element_type=jnp.float32).astype(jnp.bfloat16)
        o_ref[...] = jnp.dot(he * xt, t2_ref[...],
                             preferred_element_type=jnp.float32)

    return pl.pallas_call(
        body,
        grid=(e // tile,),
        in_specs=[
            pl.BlockSpec((tile, _LP), lambda i: (i, 0)),
            pl.BlockSpec((tile, _LP), lambda i: (i, 0)),
            pl.BlockSpec((_LP, ka), lambda i: (0, 0)),
            pl.BlockSpec((_LP, ka), lambda i: (0, 0)),
            pl.BlockSpec((ka, _LP), lambda i: (0, 0)),
        ],
        out_specs=pl.BlockSpec((tile, _LP), lambda i: (i, 0)),
        out_shape=jax.ShapeDtypeStruct((e, _LP), jnp.float32),
    )(h_p, xs_p, ra, rb, t2aug)


def _node_update(agg2, out_p, res_W, conv_bias, msg_W1, msg_W2, msg_b, init):
    n = out_p.shape[0]
    d = res_W.shape[0]
    with_init = init is not None

    def body(*refs):
        if with_init:
            (agg_ref, out_ref, rw_ref, cb_ref, w1_ref, w2_ref, mb_ref,
             init_ref, o_ref) = refs
        else:
            (agg_ref, out_ref, rw_ref, cb_ref, w1_ref, w2_ref, mb_ref,
             o_ref) = refs
        ov = out_ref[:, :d]
        agg = agg_ref[0, :, :d] + agg_ref[1, :, :d]
        m = agg + jnp.dot(ov, rw_ref[...], preferred_element_type=jnp.float32)
        m = jnp.maximum(m + cb_ref[...], 0.0)
        res = jnp.dot(m, w1_ref[...], preferred_element_type=jnp.float32)
        res = res + jnp.dot(ov, w2_ref[...], preferred_element_type=jnp.float32)
        res = res + mb_ref[...]
        if with_init:
            o_ref[...] = res + init_ref[...]
        else:
            o_ref[...] = _pad_cols(res, _LP).astype(jnp.bfloat16)

    args = [agg2, out_p, res_W, conv_bias, msg_W1, msg_W2, msg_b]
    if with_init:
        args.append(init)
    out_sd = (jax.ShapeDtypeStruct((n, d), jnp.float32) if with_init
              else jax.ShapeDtypeStruct((n, _LP), jnp.bfloat16))
    return pl.pallas_call(body, out_shape=out_sd)(*args)


def _make_gather(n, e, dtype):
    ew = e // _NW
    batch = _Q * _C
    g_outer = ew // batch
    j_chunks = ew // _C
    mesh = plsc.VectorSubcoreMesh(core_axis_name="c", subcore_axis_name="s")

    @functools.partial(
        pl.kernel,
        out_type=jax.ShapeDtypeStruct((e, _LP), dtype),
        mesh=mesh,
        compiler_params=pltpu.CompilerParams(use_tc_tiling_on_sc=False),
        scratch_types=[
            pltpu.VMEM((j_chunks, _C), jnp.int32),
            pltpu.VMEM((batch, _LP), dtype),
            pltpu.SemaphoreType.DMA,
        ],
    )
    def gath(table_hbm, idx_hbm, out_hbm, idx_v, rows_v, sem):
        cid = lax.axis_index("c")
        sid = lax.axis_index("s")
        w = cid * _NS + sid
        pltpu.sync_copy(idx_hbm.at[w], idx_v)

        def ibody(i, carry):
            handles = [
                pltpu.async_copy(
                    table_hbm.at[idx_v.at[i * _Q + q]],
                    rows_v.at[pl.ds(q * _C, _C)],
                    sem,
                )
                for q in range(_Q)
            ]
            for hd in handles:
                hd.wait()
            pltpu.sync_copy(rows_v, out_hbm.at[pl.ds(w * ew + i * batch, batch)])
            return carry

        lax.fori_loop(0, g_outer, ibody, 0)

    return gath


def _make_scatter(n, e):
    ew = e // _NW
    batch = _Q * _C
    g_outer = ew // batch
    j_chunks = ew // _C
    rows_per_tile = n // _NS
    mesh = plsc.VectorSubcoreMesh(core_axis_name="c", subcore_axis_name="s")

    @functools.partial(
        pl.kernel,
        out_type=jax.ShapeDtypeStruct((_NC, n, _LP), jnp.float32),
        mesh=mesh,
        compiler_params=pltpu.CompilerParams(use_tc_tiling_on_sc=False),
        scratch_types=[
            pltpu.VMEM((j_chunks, _C), jnp.int32),
            pltpu.VMEM((batch, _WS), jnp.float32),
            pltpu.VMEM((rows_per_tile, _WS), jnp.float32),
            pltpu.VMEM_SHARED((n, _WS), jnp.float32),
        ],
    )
    def scat(msg_hbm, dst_hbm, zeros_hbm, out_hbm, idx_v, msg_v, stage_v, agg_sh):
        cid = lax.axis_index("c")
        sid = lax.axis_index("s")
        w = cid * _NS + sid
        r0 = sid * rows_per_tile
        pltpu.sync_copy(zeros_hbm.at[pl.ds(r0, rows_per_tile)], stage_v)
        pltpu.sync_copy(stage_v, agg_sh.at[pl.ds(r0, rows_per_tile)])
        plsc.subcore_barrier()
        pltpu.sync_copy(dst_hbm.at[w], idx_v)

        def ibody(i, carry):
            pltpu.sync_copy(
                msg_hbm.at[pl.ds(w * ew + i * batch, batch), pl.ds(0, _WS)],
                msg_v,
            )
            for q in range(_Q):
                pltpu.sync_copy(
                    msg_v.at[pl.ds(q * _C, _C)],
                    agg_sh.at[idx_v.at[i * _Q + q]],
                    add=True,
                )
            return carry

        lax.fori_loop(0, g_outer, ibody, 0)
        plsc.subcore_barrier()
        pltpu.sync_copy(agg_sh.at[pl.ds(r0, rows_per_tile)], stage_v)
        pltpu.sync_copy(
            stage_v,
            out_hbm.at[cid, pl.ds(r0, rows_per_tile), pl.ds(0, _WS)],
        )

    return scat


def kernel(n_feat, edge_index, e_feat, lin0_W, lin0_b, eW1, eb1, eW2, eb2,
           res_W, conv_bias, msg_W, msg_b):
    n, d = n_feat.shape
    e = e_feat.shape[0]

    ew = e // _NW
    j_chunks = ew // _C
    assert ew * _NW == e and j_chunks * _C == ew and ew % (_Q * _C) == 0
    assert n % _NS == 0

    src3 = edge_index[0].reshape(_NW, j_chunks, _C)
    dst3 = edge_index[1].reshape(_NW, j_chunks, _C)

    ka = -(-(d * d + d) // _LP) * _LP
    t2aug = jnp.zeros((ka, _LP), jnp.float32)
    t2aug = t2aug.at[: d * d, :d].set(eW2.reshape(d * d, d))
    t2aug = t2aug.at[d * d: d * d + d, :d].set(eb2.reshape(d, d))

    t2aug16 = t2aug.astype(jnp.bfloat16)

    w1 = msg_W[:d]
    w2 = msg_W[d:]
    zeros_n = jnp.zeros((n, _WS), jnp.float32)

    gather = _make_gather(n, e, jnp.bfloat16)
    scatter = _make_scatter(n, e)

    out_p = _dense_relu_pad(n_feat, lin0_W, lin0_b.reshape(1, d))
    h_p = _edge_h(e_feat, eW1, eb1.reshape(1, d), tile=8000)

    for step in range(_STEPS):
        xs_p = gather(out_p, src3)
        msg_p = _edge_messages(h_p, xs_p, t2aug16, d, tile=1600)
        agg2 = scatter(msg_p, dst3, zeros_n)
        out_p = _node_update(
            agg2, out_p, res_W, conv_bias.reshape(1, d), w1, w2,
            msg_b.reshape(1, d),
            n_feat if step == _STEPS - 1 else None,
        )
    return out_p


# speedup vs baseline: 1.2009x; 1.2009x over previous
import functools

import jax
import jax.numpy as jnp
import numpy as np
from jax import lax
from jax.experimental import pallas as pl
from jax.experimental.pallas import tpu as pltpu
from jax.experimental.pallas import tpu_sc as plsc

_NC = 2
_NS = 16
_NW = _NC * _NS
_STEPS = 3
_LP = 128
_WS = 48
_C = 125
_Q = 5


def _pad_cols(x, width):
    n = x.shape[0]
    return jnp.concatenate([x, jnp.zeros((n, width - x.shape[1]), x.dtype)], axis=1)


def _dense_relu_pad(x, w, b):
    n, _ = x.shape

    def body(x_ref, w_ref, b_ref, o_ref):
        acc = jnp.dot(x_ref[...], w_ref[...], preferred_element_type=jnp.float32)
        o_ref[...] = _pad_cols(jnp.maximum(acc + b_ref[...], 0.0), _LP)

    return pl.pallas_call(
        body,
        out_shape=jax.ShapeDtypeStruct((n, _LP), jnp.float32),
    )(x, w, b)


def _edge_h(e_feat, eW1, eb1, tile):
    e, de = e_feat.shape
    d = eW1.shape[1]

    def body(ef_ref, w_ref, b_ref, o_ref):
        acc = jnp.dot(ef_ref[...], w_ref[...], preferred_element_type=jnp.float32)
        h = jnp.maximum(acc + b_ref[...], 0.0)
        ones = jnp.ones((tile, 1), jnp.float32)
        o_ref[...] = jnp.concatenate(
            [h, ones, jnp.zeros((tile, _LP - d - 1), jnp.float32)],
            axis=1)

    return pl.pallas_call(
        body,
        grid=(e // tile,),
        in_specs=[
            pl.BlockSpec((tile, de), lambda i: (i, 0)),
            pl.BlockSpec((de, d), lambda i: (0, 0)),
            pl.BlockSpec((1, d), lambda i: (0, 0)),
        ],
        out_specs=pl.BlockSpec((tile, _LP), lambda i: (i, 0)),
        out_shape=jax.ShapeDtypeStruct((e, _LP), jnp.float32),
    )(e_feat, eW1, eb1)


def _edge_messages(h_p, xs_p, t2aug, d, tile):
    e = h_p.shape[0]
    ka = t2aug.shape[0]

    ra_np = np.zeros((_LP, ka), np.float32)
    rb_np = np.zeros((_LP, ka), np.float32)
    eye_d = np.eye(d, dtype=np.float32)
    for k in range(d):
        ra_np[k, k * d:(k + 1) * d] = 1.0
        rb_np[:d, k * d:(k + 1) * d] = eye_d
    ra_np[d, d * d: d * d + d] = 1.0
    rb_np[:d, d * d: d * d + d] = eye_d
    ra = jnp.asarray(ra_np, dtype=jnp.bfloat16)
    rb = jnp.asarray(rb_np, dtype=jnp.bfloat16)

    def body(h_ref, xs_ref, ra_ref, rb_ref, t2_ref, o_ref):
        h16 = h_ref[...].astype(jnp.bfloat16)
        x16 = xs_ref[...].astype(jnp.bfloat16)
        he = jnp.dot(h16, ra_ref[...], preferred_element_type=jnp.float32)
        xt = jnp.dot(x16, rb_ref[...], preferred_element_type=jnp.float32)
        o_ref[...] = jnp.dot((he * xt).astype(jnp.bfloat16), t2_ref[...],
                             preferred_element_type=jnp.float32)

    return pl.pallas_call(
        body,
        grid=(e // tile,),
        in_specs=[
            pl.BlockSpec((tile, _LP), lambda i: (i, 0)),
            pl.BlockSpec((tile, _LP), lambda i: (i, 0)),
            pl.BlockSpec((_LP, ka), lambda i: (0, 0)),
            pl.BlockSpec((_LP, ka), lambda i: (0, 0)),
            pl.BlockSpec((ka, _LP), lambda i: (0, 0)),
        ],
        out_specs=pl.BlockSpec((tile, _LP), lambda i: (i, 0)),
        out_shape=jax.ShapeDtypeStruct((e, _LP), jnp.float32),
    )(h_p, xs_p, ra, rb, t2aug)


def _node_update(agg2, out_p, res_W, conv_bias, msg_W1, msg_W2, msg_b, init):
    n = out_p.shape[0]
    d = res_W.shape[0]
    with_init = init is not None

    def body(*refs):
        if with_init:
            (agg_ref, out_ref, rw_ref, cb_ref, w1_ref, w2_ref, mb_ref,
             init_ref, o_ref) = refs
        else:
            (agg_ref, out_ref, rw_ref, cb_ref, w1_ref, w2_ref, mb_ref,
             o_ref) = refs
        ov = out_ref[:, :d]
        agg = agg_ref[0, :, :d] + agg_ref[1, :, :d]
        m = agg + jnp.dot(ov, rw_ref[...], preferred_element_type=jnp.float32)
        m = jnp.maximum(m + cb_ref[...], 0.0)
        res = jnp.dot(m, w1_ref[...], preferred_element_type=jnp.float32)
        res = res + jnp.dot(ov, w2_ref[...], preferred_element_type=jnp.float32)
        res = res + mb_ref[...]
        if with_init:
            o_ref[...] = res + init_ref[...]
        else:
            o_ref[...] = _pad_cols(res, _LP)

    args = [agg2, out_p, res_W, conv_bias, msg_W1, msg_W2, msg_b]
    if with_init:
        args.append(init)
    width = d if with_init else _LP
    return pl.pallas_call(
        body,
        out_shape=jax.ShapeDtypeStruct((n, width), jnp.float32),
    )(*args)


def _make_gather(n, e, dtype):
    ew = e // _NW
    batch = _Q * _C
    g_outer = ew // batch
    j_chunks = ew // _C
    mesh = plsc.VectorSubcoreMesh(core_axis_name="c", subcore_axis_name="s")

    @functools.partial(
        pl.kernel,
        out_type=jax.ShapeDtypeStruct((e, _LP), dtype),
        mesh=mesh,
        compiler_params=pltpu.CompilerParams(use_tc_tiling_on_sc=False),
        scratch_types=[
            pltpu.VMEM((j_chunks, _C), jnp.int32),
            pltpu.VMEM((batch, _LP), dtype),
            pltpu.SemaphoreType.DMA,
        ],
    )
    def gath(table_hbm, idx_hbm, out_hbm, idx_v, rows_v, sem):
        cid = lax.axis_index("c")
        sid = lax.axis_index("s")
        w = cid * _NS + sid
        pltpu.sync_copy(idx_hbm.at[w], idx_v)

        def ibody(i, carry):
            handles = [
                pltpu.async_copy(
                    table_hbm.at[idx_v.at[i * _Q + q]],
                    rows_v.at[pl.ds(q * _C, _C)],
                    sem,
                )
                for q in range(_Q)
            ]
            for hd in handles:
                hd.wait()
            pltpu.sync_copy(rows_v, out_hbm.at[pl.ds(w * ew + i * batch, batch)])
            return carry

        lax.fori_loop(0, g_outer, ibody, 0)

    return gath


def _make_scatter(n, e):
    ew = e // _NW
    batch = _Q * _C
    g_outer = ew // batch
    j_chunks = ew // _C
    rows_per_tile = n // _NS
    mesh = plsc.VectorSubcoreMesh(core_axis_name="c", subcore_axis_name="s")

    @functools.partial(
        pl.kernel,
        out_type=jax.ShapeDtypeStruct((_NC, n, _LP), jnp.float32),
        mesh=mesh,
        compiler_params=pltpu.CompilerParams(use_tc_tiling_on_sc=False),
        scratch_types=[
            pltpu.VMEM((j_chunks, _C), jnp.int32),
            pltpu.VMEM((batch, _WS), jnp.float32),
            pltpu.VMEM((rows_per_tile, _WS), jnp.float32),
            pltpu.VMEM_SHARED((n, _WS), jnp.float32),
        ],
    )
    def scat(msg_hbm, dst_hbm, zeros_hbm, out_hbm, idx_v, msg_v, stage_v, agg_sh):
        cid = lax.axis_index("c")
        sid = lax.axis_index("s")
        w = cid * _NS + sid
        r0 = sid * rows_per_tile
        pltpu.sync_copy(zeros_hbm.at[pl.ds(r0, rows_per_tile)], stage_v)
        pltpu.sync_copy(stage_v, agg_sh.at[pl.ds(r0, rows_per_tile)])
        plsc.subcore_barrier()
        pltpu.sync_copy(dst_hbm.at[w], idx_v)

        def ibody(i, carry):
            pltpu.sync_copy(
                msg_hbm.at[pl.ds(w * ew + i * batch, batch), pl.ds(0, _WS)],
                msg_v,
            )
            for q in range(_Q):
                pltpu.sync_copy(
                    msg_v.at[pl.ds(q * _C, _C)],
                    agg_sh.at[idx_v.at[i * _Q + q]],
                    add=True,
                )
            return carry

        lax.fori_loop(0, g_outer, ibody, 0)
        plsc.subcore_barrier()
        pltpu.sync_copy(agg_sh.at[pl.ds(r0, rows_per_tile)], stage_v)
        pltpu.sync_copy(
            stage_v,
            out_hbm.at[cid, pl.ds(r0, rows_per_tile), pl.ds(0, _WS)],
        )

    return scat


def kernel(n_feat, edge_index, e_feat, lin0_W, lin0_b, eW1, eb1, eW2, eb2,
           res_W, conv_bias, msg_W, msg_b):
    n, d = n_feat.shape
    e = e_feat.shape[0]

    ew = e // _NW
    j_chunks = ew // _C
    assert ew * _NW == e and j_chunks * _C == ew and ew % (_Q * _C) == 0
    assert n % _NS == 0

    src3 = edge_index[0].reshape(_NW, j_chunks, _C)
    dst3 = edge_index[1].reshape(_NW, j_chunks, _C)

    ka = -(-(d * d + d) // _LP) * _LP
    t2aug = jnp.zeros((ka, _LP), jnp.float32)
    t2aug = t2aug.at[: d * d, :d].set(eW2.reshape(d * d, d))
    t2aug = t2aug.at[d * d: d * d + d, :d].set(eb2.reshape(d, d))

    t2aug16 = t2aug.astype(jnp.bfloat16)

    w1 = msg_W[:d]
    w2 = msg_W[d:]
    zeros_n = jnp.zeros((n, _WS), jnp.float32)

    gather = _make_gather(n, e, jnp.float32)
    scatter = _make_scatter(n, e)

    out_p = _dense_relu_pad(n_feat, lin0_W, lin0_b.reshape(1, d))
    h_p = _edge_h(e_feat, eW1, eb1.reshape(1, d), tile=8000)

    for step in range(_STEPS):
        xs_p = gather(out_p, src3)
        msg_p = _edge_messages(h_p, xs_p, t2aug16, d, tile=1600)
        agg2 = scatter(msg_p, dst3, zeros_n)
        out_p = _node_update(
            agg2, out_p, res_W, conv_bias.reshape(1, d), w1, w2,
            msg_b.reshape(1, d),
            n_feat if step == _STEPS - 1 else None,
        )
    return out_p


# speedup vs baseline: 1.2053x; 1.0036x over previous
import functools

import jax
import jax.numpy as jnp
import numpy as np
from jax import lax
from jax.experimental import pallas as pl
from jax.experimental.pallas import tpu as pltpu
from jax.experimental.pallas import tpu_sc as plsc

_NC = 2
_NS = 16
_NW = _NC * _NS
_STEPS = 3
_LP = 128
_WS = 48
_C = 125
_Q = 5


def _pad_cols(x, width):
    n = x.shape[0]
    return jnp.concatenate([x, jnp.zeros((n, width - x.shape[1]), x.dtype)], axis=1)


def _dense_relu_pad(x, w, b):
    n, _ = x.shape

    def body(x_ref, w_ref, b_ref, o_ref):
        acc = jnp.dot(x_ref[...], w_ref[...], preferred_element_type=jnp.float32)
        o_ref[...] = _pad_cols(jnp.maximum(acc + b_ref[...], 0.0), _LP)

    return pl.pallas_call(
        body,
        out_shape=jax.ShapeDtypeStruct((n, _LP), jnp.float32),
    )(x, w, b)


def _edge_h(e_feat, eW1, eb1, tile):
    e, de = e_feat.shape
    d = eW1.shape[1]

    def body(ef_ref, w_ref, b_ref, o_ref):
        acc = jnp.dot(ef_ref[...], w_ref[...], preferred_element_type=jnp.float32)
        h = jnp.maximum(acc + b_ref[...], 0.0)
        ones = jnp.ones((tile, 1), jnp.float32)
        o_ref[...] = jnp.concatenate(
            [h, ones, jnp.zeros((tile, _LP - d - 1), jnp.float32)],
            axis=1)

    return pl.pallas_call(
        body,
        grid=(e // tile,),
        in_specs=[
            pl.BlockSpec((tile, de), lambda i: (i, 0)),
            pl.BlockSpec((de, d), lambda i: (0, 0)),
            pl.BlockSpec((1, d), lambda i: (0, 0)),
        ],
        out_specs=pl.BlockSpec((tile, _LP), lambda i: (i, 0)),
        out_shape=jax.ShapeDtypeStruct((e, _LP), jnp.float32),
    )(e_feat, eW1, eb1)


def _edge_messages(h_p, xs_p, t2aug, d, tile):
    e = h_p.shape[0]
    ka = t2aug.shape[0]

    ra_np = np.zeros((_LP, ka), np.float32)
    rb_np = np.zeros((_LP, ka), np.float32)
    eye_d = np.eye(d, dtype=np.float32)
    for k in range(d):
        ra_np[k, k * d:(k + 1) * d] = 1.0
        rb_np[:d, k * d:(k + 1) * d] = eye_d
    ra_np[d, d * d: d * d + d] = 1.0
    rb_np[:d, d * d: d * d + d] = eye_d
    ra = jnp.asarray(ra_np, dtype=jnp.bfloat16)
    rb = jnp.asarray(rb_np, dtype=jnp.bfloat16)

    def body(h_ref, xs_ref, ra_ref, rb_ref, t2_ref, o_ref):
        h16 = h_ref[...].astype(jnp.bfloat16)
        x16 = xs_ref[...].astype(jnp.bfloat16)
        he = jnp.dot(h16, ra_ref[...], preferred_element_type=jnp.float32)
        xt = jnp.dot(x16, rb_ref[...], preferred_element_type=jnp.float32)
        o_ref[...] = jnp.dot((he * xt).astype(jnp.bfloat16), t2_ref[...],
                             preferred_element_type=jnp.float32)

    return pl.pallas_call(
        body,
        grid=(e // tile,),
        in_specs=[
            pl.BlockSpec((tile, _LP), lambda i: (i, 0)),
            pl.BlockSpec((tile, _LP), lambda i: (i, 0)),
            pl.BlockSpec((_LP, ka), lambda i: (0, 0)),
            pl.BlockSpec((_LP, ka), lambda i: (0, 0)),
            pl.BlockSpec((ka, _LP), lambda i: (0, 0)),
        ],
        out_specs=pl.BlockSpec((tile, _LP), lambda i: (i, 0)),
        out_shape=jax.ShapeDtypeStruct((e, _LP), jnp.float32),
    )(h_p, xs_p, ra, rb, t2aug)


def _node_update(agg2, out_p, res_W, conv_bias, msg_W1, msg_W2, msg_b, init):
    n = out_p.shape[0]
    d = res_W.shape[0]
    with_init = init is not None

    def body(*refs):
        if with_init:
            (agg_ref, out_ref, rw_ref, cb_ref, w1_ref, w2_ref, mb_ref,
             init_ref, o_ref) = refs
        else:
            (agg_ref, out_ref, rw_ref, cb_ref, w1_ref, w2_ref, mb_ref,
             o_ref) = refs
        ov = out_ref[:, :d]
        agg = agg_ref[0, :, :d] + agg_ref[1, :, :d]
        m = agg + jnp.dot(ov, rw_ref[...], preferred_element_type=jnp.float32)
        m = jnp.maximum(m + cb_ref[...], 0.0)
        res = jnp.dot(m, w1_ref[...], preferred_element_type=jnp.float32)
        res = res + jnp.dot(ov, w2_ref[...], preferred_element_type=jnp.float32)
        res = res + mb_ref[...]
        if with_init:
            o_ref[...] = res + init_ref[...]
        else:
            o_ref[...] = _pad_cols(res, _LP)

    args = [agg2, out_p, res_W, conv_bias, msg_W1, msg_W2, msg_b]
    if with_init:
        args.append(init)
    width = d if with_init else _LP
    return pl.pallas_call(
        body,
        out_shape=jax.ShapeDtypeStruct((n, width), jnp.float32),
    )(*args)


def _make_gather(n, e, dtype):
    ew = e // _NW
    gq = 8
    batch = gq * _C
    g_outer = ew // batch
    j_chunks = ew // _C
    mesh = plsc.VectorSubcoreMesh(core_axis_name="c", subcore_axis_name="s")

    @functools.partial(
        pl.kernel,
        out_type=jax.ShapeDtypeStruct((e, _LP), dtype),
        mesh=mesh,
        compiler_params=pltpu.CompilerParams(use_tc_tiling_on_sc=True),
        scratch_types=[
            pltpu.VMEM((gq, _C), jnp.int32),
            pltpu.VMEM((batch, _LP), dtype),
            pltpu.SemaphoreType.DMA,
        ],
    )
    def gath(table_hbm, idx_hbm, out_hbm, idx_v, rows_v, sem):
        cid = lax.axis_index("c")
        sid = lax.axis_index("s")
        w = cid * _NS + sid

        def ibody(i, carry):
            pltpu.sync_copy(idx_hbm.at[w, pl.ds(i * gq, gq)], idx_v)
            handles = [
                pltpu.async_copy(
                    table_hbm.at[idx_v.at[q]],
                    rows_v.at[pl.ds(q * _C, _C)],
                    sem,
                )
                for q in range(gq)
            ]
            for hd in handles:
                hd.wait()
            pltpu.sync_copy(rows_v, out_hbm.at[pl.ds(w * ew + i * batch, batch)])
            return carry

        lax.fori_loop(0, g_outer, ibody, 0)

    return gath


def _make_scatter(n, e):
    ew = e // _NW
    batch = _Q * _C
    g_outer = ew // batch
    j_chunks = ew // _C
    rows_per_tile = n // _NS
    mesh = plsc.VectorSubcoreMesh(core_axis_name="c", subcore_axis_name="s")

    @functools.partial(
        pl.kernel,
        out_type=jax.ShapeDtypeStruct((_NC, n, _LP), jnp.float32),
        mesh=mesh,
        compiler_params=pltpu.CompilerParams(use_tc_tiling_on_sc=False),
        scratch_types=[
            pltpu.VMEM((j_chunks, _C), jnp.int32),
            pltpu.VMEM((batch, _WS), jnp.float32),
            pltpu.VMEM((rows_per_tile, _WS), jnp.float32),
            pltpu.VMEM_SHARED((n, _WS), jnp.float32),
        ],
    )
    def scat(msg_hbm, dst_hbm, zeros_hbm, out_hbm, idx_v, msg_v, stage_v, agg_sh):
        cid = lax.axis_index("c")
        sid = lax.axis_index("s")
        w = cid * _NS + sid
        r0 = sid * rows_per_tile
        pltpu.sync_copy(zeros_hbm.at[pl.ds(r0, rows_per_tile)], stage_v)
        pltpu.sync_copy(stage_v, agg_sh.at[pl.ds(r0, rows_per_tile)])
        plsc.subcore_barrier()
        pltpu.sync_copy(dst_hbm.at[w], idx_v)

        def ibody(i, carry):
            pltpu.sync_copy(
                msg_hbm.at[pl.ds(w * ew + i * batch, batch), pl.ds(0, _WS)],
                msg_v,
            )
            for q in range(_Q):
                pltpu.sync_copy(
                    msg_v.at[pl.ds(q * _C, _C)],
                    agg_sh.at[idx_v.at[i * _Q + q]],
                    add=True,
                )
            return carry

        lax.fori_loop(0, g_outer, ibody, 0)
        plsc.subcore_barrier()
        pltpu.sync_copy(agg_sh.at[pl.ds(r0, rows_per_tile)], stage_v)
        pltpu.sync_copy(
            stage_v,
            out_hbm.at[cid, pl.ds(r0, rows_per_tile), pl.ds(0, _WS)],
        )

    return scat


def kernel(n_feat, edge_index, e_feat, lin0_W, lin0_b, eW1, eb1, eW2, eb2,
           res_W, conv_bias, msg_W, msg_b):
    n, d = n_feat.shape
    e = e_feat.shape[0]

    ew = e // _NW
    j_chunks = ew // _C
    assert ew * _NW == e and j_chunks * _C == ew and ew % (_Q * _C) == 0
    assert n % _NS == 0

    src3 = edge_index[0].reshape(_NW, j_chunks, _C)
    dst3 = edge_index[1].reshape(_NW, j_chunks, _C)

    ka = -(-(d * d + d) // _LP) * _LP
    t2aug = jnp.zeros((ka, _LP), jnp.float32)
    t2aug = t2aug.at[: d * d, :d].set(eW2.reshape(d * d, d))
    t2aug = t2aug.at[d * d: d * d + d, :d].set(eb2.reshape(d, d))

    t2aug16 = t2aug.astype(jnp.bfloat16)

    w1 = msg_W[:d]
    w2 = msg_W[d:]
    zeros_n = jnp.zeros((n, _WS), jnp.float32)

    gather = _make_gather(n, e, jnp.float32)
    scatter = _make_scatter(n, e)

    out_p = _dense_relu_pad(n_feat, lin0_W, lin0_b.reshape(1, d))
    h_p = _edge_h(e_feat, eW1, eb1.reshape(1, d), tile=8000)

    for step in range(_STEPS):
        xs_p = gather(out_p, src3)
        msg_p = _edge_messages(h_p, xs_p, t2aug16, d, tile=1600)
        agg2 = scatter(msg_p, dst3, zeros_n)
        out_p = _node_update(
            agg2, out_p, res_W, conv_bias.reshape(1, d), w1, w2,
            msg_b.reshape(1, d),
            n_feat if step == _STEPS - 1 else None,
        )
    return out_p
